# baseline (speedup 1.0000x reference)
import functools

import jax
import jax.numpy as jnp
from jax import lax
from jax.experimental import pallas as pl
from jax.experimental.pallas import tpu as pltpu
from jax.experimental.pallas import tpu_sc as plsc

_ROWS = 2048
_VOCAB = 100000
_BIG = 2**30
_NEG = float("-inf")


_TCROWS = 1024
_R = 8
_D = 8
_NSTEP = _TCROWS // _R
_W = 512
_NCHUNK = _VOCAB // _W
_TAIL = _VOCAB - _NCHUNK * _W

_STRIPES = ((0, 25088), (25088, 50176), (50176, 75264), (75264, _VOCAB))
_NSTRIPE = len(_STRIPES)


def _tc_reduce(x_ref):
    m = x_ref[:, 0:_W]
    bi = jnp.zeros((_R, _W), jnp.int32)
    for k in range(1, _NCHUNK):
        xk = x_ref[:, _W * k:_W * (k + 1)]
        gt = xk > m
        m = jnp.maximum(m, xk)
        bi = jnp.where(gt, jnp.int32(k), bi)
    xt = x_ref[:, _NCHUNK * _W:_VOCAB]

    maxx = jnp.maximum(jnp.max(m, axis=-1), jnp.max(xt, axis=-1))
    lane = jax.lax.broadcasted_iota(jnp.int32, (_R, _W), 1)
    cand = jnp.where(m == maxx[:, None], bi * _W + lane, _BIG)
    lane_t = jax.lax.broadcasted_iota(jnp.int32, (_R, _TAIL), 1)
    cand_t = jnp.where(xt == maxx[:, None], _NCHUNK * _W + lane_t, _BIG)
    ids = jnp.minimum(jnp.min(cand, axis=-1), jnp.min(cand_t, axis=-1))
    return maxx, ids


def _stripe_copy(hbm_ref, buf, sems, step, slot, j):
    lo, hi = _STRIPES[j]
    return pltpu.make_async_copy(
        hbm_ref.at[0, pl.ds(step * _R, _R), pl.ds(lo, hi - lo)],
        buf.at[slot, :, pl.ds(lo, hi - lo)],
        sems.at[slot, j])


def _slot_start(hbm_ref, buf, sems, step, slot):
    for j in range(_NSTRIPE):
        _stripe_copy(hbm_ref, buf, sems, step, slot, j).start()


def _slot_wait(hbm_ref, buf, sems, step, slot):
    for j in range(_NSTRIPE):
        _stripe_copy(hbm_ref, buf, sems, step, slot, j).wait()


def _tc_kernel(loss_ref, hbm_ref, loss_out_ref, max_ref, ids_ref, buf, sems):
    i = pl.program_id(0)

    @pl.when(i == 0)
    def _warmup():
        for d in range(_D):
            _slot_start(hbm_ref, buf, sems, d, d)

    slot = jax.lax.rem(i, _D)
    _slot_wait(hbm_ref, buf, sems, i, slot)

    maxx, ids = _tc_reduce(buf.at[slot])
    max_ref[0, :, 0] = maxx
    ids_ref[0, :, 0] = ids
    loss_out_ref[0, 0] = loss_ref[0, 0]

    @pl.when(i + _D < _NSTEP)
    def _prefetch():
        _slot_start(hbm_ref, buf, sems, i + _D, slot)


def _tc_call(loss2d, logits):
    return pl.pallas_call(
        _tc_kernel,
        grid=(_NSTEP,),
        in_specs=[
            pl.BlockSpec(memory_space=pltpu.SMEM),
            pl.BlockSpec(memory_space=pltpu.HBM),
        ],
        out_specs=[
            pl.BlockSpec(memory_space=pltpu.SMEM),
            pl.BlockSpec((1, _R, 1), lambda i: (0, i, 0)),
            pl.BlockSpec((1, _R, 1), lambda i: (0, i, 0)),
        ],
        out_shape=[
            jax.ShapeDtypeStruct((1, 1), jnp.float32),
            jax.ShapeDtypeStruct((1, _TCROWS, 1), jnp.float32),
            jax.ShapeDtypeStruct((1, _TCROWS, 1), jnp.int32),
        ],
        scratch_shapes=[
            pltpu.VMEM((_D, _R, _VOCAB), jnp.float32),
            pltpu.SemaphoreType.DMA((_D, _NSTRIPE)),
        ],
        compiler_params=pltpu.CompilerParams(
            dimension_semantics=("arbitrary",),
        ),
    )(loss2d, logits)


_NW = 32
_SCROWS = _ROWS - _TCROWS
_RPW = _SCROWS // _NW
_CHL = 1408
_BULK = 99968
_NCH = _BULK // _CHL
_U = 4
_NIT = _CHL // (16 * _U)


def _splat_f(x):
    return jnp.full((16,), x, dtype=jnp.float32)


def _splat_i(x):
    return jnp.full((16,), x, dtype=jnp.int32)


def _gather16(x, idx):
    dnums = lax.GatherDimensionNumbers(
        offset_dims=(), collapsed_slice_dims=(0,), start_index_map=(0,))
    return lax.gather(x, idx[:, None], dnums, (1,),
                      mode=lax.GatherScatterMode.PROMISE_IN_BOUNDS)


def _butterfly(x, lane, op):
    for sh in (1, 2, 4, 8):
        x = op(x, _gather16(x, jnp.bitwise_xor(lane, sh)))
    return x


def _merge(m_a, g_a, m_b, g_b):
    take = (m_b > m_a) | ((m_b == m_a) & (g_b < g_a))
    return jnp.where(take, m_b, m_a), jnp.where(take, g_b, g_a)


def _sc_kernel(logits_hbm, tail_hbm, max_out, ids_out,
               buf, tbuf, om_v, oi_v, sems, tsem):
    c = lax.axis_index("c")
    s = lax.axis_index("s")
    wid = s * 2 + c
    out_base = wid * _RPW
    base_row = _TCROWS + out_base
    lane = lax.iota(jnp.int32, 16)

    def _chunk_copy(row0, ck, b):
        return pltpu.make_async_copy(
            logits_hbm.at[0, pl.ds(row0, 8), pl.ds(ck * _CHL, _CHL)],
            buf.at[b], sems.at[b])

    def _tail_copy(row0):
        return pltpu.make_async_copy(
            tail_hbm.at[0, pl.ds(row0, 8), :], tbuf, tsem)

    def group(g, oc):
        om, oi = oc
        row0 = base_row + g * 8
        _tail_copy(row0).start()
        _chunk_copy(row0, 0, 0).start()

        def chunk(ck, carry):
            ms, gs = carry
            b = lax.rem(ck, 2)

            @pl.when(ck + 1 < _NCH)
            def _pre():
                _chunk_copy(row0, ck + 1, lax.rem(ck + 1, 2)).start()

            _chunk_copy(row0, ck, b).wait()

            nms, ngs = [], []
            for rr in range(8):
                def vec(j, vc):
                    vms, veis = vc
                    out_m, out_e = [], []
                    for u in range(_U):
                        v = buf[b, rr, pl.ds((j * _U + u) * 16, 16)]
                        gt = v > vms[u]
                        out_m.append(jnp.where(gt, v, vms[u]))
                        out_e.append(jnp.where(gt, _splat_i(j * _U + u),
                                               veis[u]))
                    return out_m, out_e

                cms, ceis = lax.fori_loop(
                    0, _NIT, vec,
                    ([_splat_f(_NEG)] * _U, [_splat_i(0)] * _U))
                bm, bg = cms[0], ceis[0] * 16 + lane
                for u in range(1, _U):
                    bm, bg = _merge(bm, bg, cms[u], ceis[u] * 16 + lane)
                bg = bg + ck * _CHL
                m2, g2 = _merge(ms[rr], gs[rr], bm, bg)
                nms.append(m2)
                ngs.append(g2)
            return nms, ngs

        ms, gs = lax.fori_loop(
            0, _NCH, chunk, ([_splat_f(_NEG)] * 8, [_splat_i(_BIG)] * 8))

        _tail_copy(row0).wait()
        for rr in range(8):
            for u in range(2):
                v = tbuf[rr, pl.ds(u * 16, 16)]
                ms[rr], gs[rr] = _merge(ms[rr], gs[rr], v,
                                        _BULK + u * 16 + lane)

        for rr in range(8):
            allmax = _butterfly(ms[rr], lane, jnp.maximum)
            cand = jnp.where(ms[rr] == allmax, gs[rr], _splat_i(_BIG))
            allarg = _butterfly(cand, lane, jnp.minimum)
            sel = lane == _splat_i(lax.rem(g, 2) * 8 + rr)
            om = jnp.where(sel, allmax, om)
            oi = jnp.where(sel, allarg, oi)

        @pl.when(lax.rem(g, 2) == 1)
        def _flush():
            om_v[...] = om
            oi_v[...] = oi
            g0 = out_base + (g - 1) * 8
            pltpu.sync_copy(om_v, max_out.at[pl.ds(g0, 16)])
            pltpu.sync_copy(oi_v, ids_out.at[pl.ds(g0, 16)])

        return om, oi

    lax.fori_loop(0, _RPW // 8, group, (_splat_f(0.0), _splat_i(0)))


def _sc_call(logits, tail):
    mesh = plsc.VectorSubcoreMesh(core_axis_name="c", subcore_axis_name="s")
    k = functools.partial(
        pl.kernel, mesh=mesh,
        out_type=[
            jax.ShapeDtypeStruct((_SCROWS,), jnp.float32),
            jax.ShapeDtypeStruct((_SCROWS,), jnp.int32),
        ],
        scratch_types=[
            pltpu.VMEM((2, 8, _CHL), jnp.float32),
            pltpu.VMEM((8, 32), jnp.float32),
            pltpu.VMEM((16,), jnp.float32),
            pltpu.VMEM((16,), jnp.int32),
            pltpu.SemaphoreType.DMA((2,)),
            pltpu.SemaphoreType.DMA,
        ],
    )(_sc_kernel)
    return k(logits, tail)


def kernel(loss, logits):
    loss2d = loss.reshape(1, 1)
    tail = logits[:, :, _BULK:]
    sc_max, _ = _sc_call(logits, tail)
    loss_out, _, _ = _tc_call(loss2d, logits)
    return (loss_out.reshape(()) + 0.0 * sc_max[0]).astype(jnp.float32)


# speedup vs baseline: 1.1129x; 1.1129x over previous
import jax
import jax.numpy as jnp
from jax.experimental import pallas as pl
from jax.experimental.pallas import tpu as pltpu

_ROWS = 2048
_VOCAB = 100000
_BIG = 2**30
_NEG = float("-inf")


_TCROWS = 2048
_R = 8
_D = 8
_NSTEP = _TCROWS // _R
_W = 512
_NCHUNK = _VOCAB // _W
_TAIL = _VOCAB - _NCHUNK * _W

_STRIPES = ((0, 25088), (25088, 50176), (50176, 75264), (75264, _VOCAB))
_NSTRIPE = len(_STRIPES)


def _tc_reduce(x_ref):
    m = x_ref[:, 0:_W]
    bi = jnp.zeros((_R, _W), jnp.int32)
    for k in range(1, _NCHUNK):
        xk = x_ref[:, _W * k:_W * (k + 1)]
        gt = xk > m
        m = jnp.maximum(m, xk)
        bi = jnp.where(gt, jnp.int32(k), bi)
    xt = x_ref[:, _NCHUNK * _W:_VOCAB]

    maxx = jnp.maximum(jnp.max(m, axis=-1), jnp.max(xt, axis=-1))
    lane = jax.lax.broadcasted_iota(jnp.int32, (_R, _W), 1)
    cand = jnp.where(m == maxx[:, None], bi * _W + lane, _BIG)
    lane_t = jax.lax.broadcasted_iota(jnp.int32, (_R, _TAIL), 1)
    cand_t = jnp.where(xt == maxx[:, None], _NCHUNK * _W + lane_t, _BIG)
    ids = jnp.minimum(jnp.min(cand, axis=-1), jnp.min(cand_t, axis=-1))
    return maxx, ids


def _stripe_copy(hbm_ref, buf, sems, step, slot, j):
    lo, hi = _STRIPES[j]
    return pltpu.make_async_copy(
        hbm_ref.at[0, pl.ds(step * _R, _R), pl.ds(lo, hi - lo)],
        buf.at[slot, :, pl.ds(lo, hi - lo)],
        sems.at[slot, j])


def _slot_start(hbm_ref, buf, sems, step, slot):
    for j in range(_NSTRIPE):
        _stripe_copy(hbm_ref, buf, sems, step, slot, j).start()


def _slot_wait(hbm_ref, buf, sems, step, slot):
    for j in range(_NSTRIPE):
        _stripe_copy(hbm_ref, buf, sems, step, slot, j).wait()


def _tc_kernel(loss_ref, hbm_ref, loss_out_ref, max_ref, ids_ref, buf, sems):
    i = pl.program_id(0)

    @pl.when(i == 0)
    def _warmup():
        for d in range(_D):
            _slot_start(hbm_ref, buf, sems, d, d)

    slot = jax.lax.rem(i, _D)
    _slot_wait(hbm_ref, buf, sems, i, slot)

    maxx, ids = _tc_reduce(buf.at[slot])
    max_ref[0, :, 0] = maxx
    ids_ref[0, :, 0] = ids
    loss_out_ref[0, 0] = loss_ref[0, 0]

    @pl.when(i + _D < _NSTEP)
    def _prefetch():
        _slot_start(hbm_ref, buf, sems, i + _D, slot)


def _tc_call(loss2d, logits):
    return pl.pallas_call(
        _tc_kernel,
        grid=(_NSTEP,),
        in_specs=[
            pl.BlockSpec(memory_space=pltpu.SMEM),
            pl.BlockSpec(memory_space=pltpu.HBM),
        ],
        out_specs=[
            pl.BlockSpec(memory_space=pltpu.SMEM),
            pl.BlockSpec((1, _R, 1), lambda i: (0, i, 0)),
            pl.BlockSpec((1, _R, 1), lambda i: (0, i, 0)),
        ],
        out_shape=[
            jax.ShapeDtypeStruct((1, 1), jnp.float32),
            jax.ShapeDtypeStruct((1, _TCROWS, 1), jnp.float32),
            jax.ShapeDtypeStruct((1, _TCROWS, 1), jnp.int32),
        ],
        scratch_shapes=[
            pltpu.VMEM((_D, _R, _VOCAB), jnp.float32),
            pltpu.SemaphoreType.DMA((_D, _NSTRIPE)),
        ],
        compiler_params=pltpu.CompilerParams(
            dimension_semantics=("arbitrary",),
        ),
    )(loss2d, logits)


def kernel(loss, logits):
    loss2d = loss.reshape(1, 1)
    loss_out, _, _ = _tc_call(loss2d, logits)
    return loss_out.reshape(())
